# Initial kernel scaffold; baseline (speedup 1.0000x reference)
import jax
import jax.numpy as jnp
from jax.experimental import pallas as pl


def kernel(token_ids, table):
    raise NotImplementedError("write your pallas kernel here")


# speedup vs baseline: 1.4768x; 1.4768x over previous
import functools

import jax
import jax.numpy as jnp
from jax import lax
from jax.experimental import pallas as pl
from jax.experimental.pallas import tpu as pltpu
from jax.experimental.pallas import tpu_sc as plsc

_NW = 32
_CH = 1600


def _gather_body(per_w, n_chunks, ids_hbm, table_hbm, out_hbm, idx_v, rows_v, sem):
    wid = lax.axis_index("s") * 2 + lax.axis_index("c")
    base = wid * per_w

    def body(g, carry):
        off = base + g * _CH
        pltpu.sync_copy(ids_hbm.at[pl.ds(off, _CH)], idx_v)
        pltpu.async_copy(table_hbm.at[idx_v], rows_v, sem).wait()
        pltpu.sync_copy(rows_v, out_hbm.at[pl.ds(off, _CH)])
        return carry

    lax.fori_loop(0, n_chunks, body, 0, unroll=False)


def kernel(token_ids, table):
    b, s = token_ids.shape
    _, d = table.shape
    n = b * s
    assert n % (_NW * _CH) == 0
    per_w = n // _NW
    n_chunks = per_w // _CH

    flat_ids = token_ids.reshape(n).astype(jnp.int32)
    mesh = plsc.VectorSubcoreMesh(core_axis_name="c", subcore_axis_name="s")
    k = pl.kernel(
        functools.partial(_gather_body, per_w, n_chunks),
        out_type=jax.ShapeDtypeStruct((n, d), jnp.float32),
        mesh=mesh,
        scratch_types=[
            pltpu.VMEM((_CH,), jnp.int32),
            pltpu.VMEM((_CH, d), jnp.float32),
            pltpu.SemaphoreType.DMA,
        ],
        compiler_params=pltpu.CompilerParams(use_tc_tiling_on_sc=False),
    )
    out = k(flat_ids, table)
    return out.reshape(b, s, d)


# speedup vs baseline: 1.4950x; 1.0123x over previous
import functools

import jax
import jax.numpy as jnp
from jax import lax
from jax.experimental import pallas as pl
from jax.experimental.pallas import tpu as pltpu
from jax.experimental.pallas import tpu_sc as plsc

_NW = 32
_CH = 1280


def _gather_body(per_w, n_pairs, ids_hbm, table_hbm, out_hbm,
                 idx_v, rows0, rows1, gs0, gs1, os0, os1):
    ch = _CH
    wid = lax.axis_index("s") * 2 + lax.axis_index("c")
    base = wid * per_w
    pltpu.sync_copy(ids_hbm.at[pl.ds(base, per_w)], idx_v)

    def g_copy(g, rows, sem):
        return pltpu.make_async_copy(
            table_hbm.at[idx_v.at[pl.ds(g * ch, ch)]], rows, sem)

    def s_copy(g, rows, sem):
        return pltpu.make_async_copy(
            rows, out_hbm.at[pl.ds(base + g * ch, ch)], sem)

    g_copy(0, rows0, gs0).start()

    def body(i, carry):
        a = 2 * i
        g_copy(a, rows0, gs0).wait()
        s_copy(a, rows0, os0).start()

        @pl.when(i > 0)
        def _():
            s_copy(a - 1, rows1, os1).wait()

        g_copy(a + 1, rows1, gs1).start()
        g_copy(a + 1, rows1, gs1).wait()
        s_copy(a + 1, rows1, os1).start()

        @pl.when(i + 1 < n_pairs)
        def _():
            s_copy(a, rows0, os0).wait()
            g_copy(a + 2, rows0, gs0).start()

        return carry

    lax.fori_loop(0, n_pairs, body, 0, unroll=False)
    s_copy(0, rows0, os0).wait()
    s_copy(0, rows1, os1).wait()


def kernel(token_ids, table):
    b, s = token_ids.shape
    _, d = table.shape
    n = b * s
    assert n % (_NW * 2 * _CH) == 0
    per_w = n // _NW
    n_pairs = per_w // (2 * _CH)

    flat_ids = token_ids.reshape(n).astype(jnp.int32)
    mesh = plsc.VectorSubcoreMesh(core_axis_name="c", subcore_axis_name="s")
    k = pl.kernel(
        functools.partial(_gather_body, per_w, n_pairs),
        out_type=jax.ShapeDtypeStruct((n, d), jnp.float32),
        mesh=mesh,
        scratch_types=[
            pltpu.VMEM((per_w,), jnp.int32),
            pltpu.VMEM((_CH, d), jnp.float32),
            pltpu.VMEM((_CH, d), jnp.float32),
            pltpu.SemaphoreType.DMA,
            pltpu.SemaphoreType.DMA,
            pltpu.SemaphoreType.DMA,
            pltpu.SemaphoreType.DMA,
        ],
        compiler_params=pltpu.CompilerParams(use_tc_tiling_on_sc=False),
    )
    out = k(flat_ids, table)
    return out.reshape(b, s, d)
